# Initial kernel scaffold; baseline (speedup 1.0000x reference)
import jax
import jax.numpy as jnp
from jax.experimental import pallas as pl


def kernel(x):
    raise NotImplementedError("write your pallas kernel here")


# speedup vs baseline: 4.8573x; 4.8573x over previous
import jax
import jax.numpy as jnp
from jax import lax
from jax.experimental import pallas as pl
from jax.experimental.pallas import tpu as pltpu
from jax.experimental.pallas import tpu_sc as plsc

B0, B1, N = 4, 4096, 2048
ROWS = B0 * B1
K = (N - 1) // 2
NW = 32
ROWS_PER_W = ROWS // NW
RB = 32
L = 16
UNROLL = 8


def _median_body(x_hbm, out_hbm, x_vmem, out_vmem):
    c = lax.axis_index("c")
    s = lax.axis_index("s")
    wid = s * 2 + c
    row0 = wid * ROWS_PER_W

    kvec = jnp.full((L,), K, jnp.int32)
    zero = jnp.zeros((L,), jnp.int32)
    top_bit = jnp.full((L,), -(2 ** 31), jnp.int32)
    mant_mask = jnp.full((L,), 0x7FFFFFFF, jnp.int32)
    lane = lax.iota(jnp.int32, L)
    lane0 = lane == zero

    def key_to_float(key):
        fbits = jnp.where(key < zero, key & mant_mask, ~key)
        return lax.bitcast_convert_type(fbits, jnp.float32)

    def block_body(blk, carry):
        base = blk * RB
        pltpu.sync_copy(x_hbm.at[pl.ds((row0 + base) * N, RB * N)], x_vmem)

        def row_body(r, carry):
            off = r * N

            def bit_body(_, pb):
                prefix, bitv = pb
                cand = prefix | bitv
                t = key_to_float(cand)

                def chunk(j, cnt):
                    e = off + j * (UNROLL * L)
                    for u in range(UNROLL):
                        v = x_vmem[pl.ds(e + u * L, L)]
                        cnt = cnt + plsc.all_reduce_population_count(v < t)
                    return cnt

                cnt = lax.fori_loop(0, N // (UNROLL * L), chunk, zero)
                prefix = jnp.where(cnt <= kvec, cand, prefix)
                return prefix, lax.shift_right_logical(bitv, 1)

            prefix, _ = lax.fori_loop(0, 32, bit_body, (zero, top_bit))
            val = key_to_float(prefix)
            plsc.store_scatter(out_vmem, [jnp.full((L,), base + r, jnp.int32)],
                               val, mask=lane0)
            return carry

        return lax.fori_loop(0, RB, row_body, carry)

    lax.fori_loop(0, ROWS_PER_W // RB, block_body, 0)
    pltpu.sync_copy(out_vmem, out_hbm.at[pl.ds(row0, ROWS_PER_W)])


def _median_call(xflat):
    return pl.kernel(
        _median_body,
        out_type=jax.ShapeDtypeStruct((ROWS,), jnp.float32),
        mesh=plsc.VectorSubcoreMesh(core_axis_name="c", subcore_axis_name="s"),
        scratch_types=[
            pltpu.VMEM((RB * N,), jnp.float32),
            pltpu.VMEM((ROWS_PER_W,), jnp.float32),
        ],
        compiler_params=pltpu.CompilerParams(needs_layout_passes=False),
    )(xflat)


@jax.jit
def kernel(x):
    out = _median_call(x.reshape(ROWS * N))
    return out.reshape(B0, B1)
